# baseline (speedup 1.0000x reference)
import jax
import jax.numpy as jnp
from jax.experimental import pallas as pl
from jax.experimental.pallas import tpu as pltpu

_N = 5000
_B = 256
_NB = (_N + _B - 1) // _B
_NPAD = _NB * _B
_SCORE_THRESH = 0.05
_NMS_THRESH = 0.5
_MAX_DET = 100


def _iou_mask(ax1, ay1, ax2, ay2, bx1, by1, bx2, by2):
    ix1 = jnp.maximum(ax1, bx1)
    iy1 = jnp.maximum(ay1, by1)
    ix2 = jnp.minimum(ax2, bx2)
    iy2 = jnp.minimum(ay2, by2)
    iw = jnp.maximum(ix2 - ix1, 0.0)
    ih = jnp.maximum(iy2 - iy1, 0.0)
    inter = iw * ih
    area_a = (ax2 - ax1) * (ay2 - ay1)
    area_b = (bx2 - bx1) * (by2 - by1)
    union = area_a + area_b - inter
    iou = inter / jnp.maximum(union, 1e-9)
    return iou > _NMS_THRESH


def _nms_kernel(x1c, y1c, x2c, y2c, sc,
                out_ref,
                keep_ref):
    f32 = jnp.float32
    hi = jax.lax.Precision.HIGHEST

    def row(ref, bi):
        return ref[pl.ds(bi, 1), :]

    def col(ref, bi):
        return ref[pl.ds(bi * _B, _B), :]

    ii = jax.lax.broadcasted_iota(jnp.int32, (_B, _B), 0)
    jj = jax.lax.broadcasted_iota(jnp.int32, (_B, _B), 1)
    upper = (ii < jj)
    upper_f = upper.astype(f32)
    eye = (ii == jj).astype(f32)

    def to_row(c):
        return jax.lax.dot_general(c, eye, (((0,), (0,)), ((), ())),
                                   preferred_element_type=f32, precision=hi)

    def matvec(k, m):
        return jnp.dot(k, m, preferred_element_type=f32)

    def main_cond(st):
        bi, cnt = st
        return jnp.logical_and(bi < _NB, cnt < jnp.float32(_MAX_DET))

    def main_body(st):
        bi, cnt = st
        bx1, by1 = to_row(col(x1c, bi)), to_row(col(y1c, bi))
        bx2, by2 = to_row(col(x2c, bi)), to_row(col(y2c, bi))
        sb = to_row(col(sc, bi))
        v = (sb > _SCORE_THRESH).astype(f32)

        def cross(bj, v):
            m = _iou_mask(col(x1c, bj), col(y1c, bj), col(x2c, bj), col(y2c, bj),
                          bx1, by1, bx2, by2)
            krow = row(keep_ref, bj)
            supp = matvec(krow, m.astype(f32))
            return jnp.where(supp > 0.0, 0.0, v)

        v = jax.lax.fori_loop(0, bi, cross, v)

        m = _iou_mask(col(x1c, bi), col(y1c, bi), col(x2c, bi), col(y2c, bi),
                      bx1, by1, bx2, by2)
        mf = jnp.where(upper, m.astype(f32), 0.0)
        vf = v

        def conv_cond(cs):
            _, changed = cs
            return changed

        def conv_body(cs):
            k, _ = cs
            supp = matvec(k, mf) > 0.0
            k_new = jnp.where(supp, 0.0, vf)
            return k_new, jnp.any(k_new != k)

        k, _ = jax.lax.while_loop(conv_cond, conv_body, (vf, jnp.bool_(True)))

        keep_ref[pl.ds(bi, 1), :] = k
        return bi + 1, cnt + jnp.sum(k)

    nblk, cnt = jax.lax.while_loop(main_cond, main_body,
                                   (jnp.int32(0), jnp.float32(0.0)))

    kcap = jnp.minimum(cnt, jnp.float32(_MAX_DET))
    iom = jax.lax.broadcasted_iota(jnp.int32, (_MAX_DET, _B), 0)
    ioj = jax.lax.broadcasted_iota(jnp.int32, (1, _B), 1)
    ones_col = jnp.ones((_B, 1), f32)

    def sel_body(bj, carry):
        kept_before, nk_before, acc4, acc1 = carry
        k = row(keep_ref, bj)
        real = ((bj * _B + ioj) < _N).astype(f32)
        nk = (1.0 - k) * real

        pk = matvec(k, upper_f)
        pn = matvec(nk, upper_f)
        slot = jnp.where(k > 0.0, kept_before + pk, kcap + nk_before + pn)
        sel = jnp.logical_and(jnp.logical_or(k > 0.0, nk > 0.0),
                              slot < jnp.float32(_MAX_DET))
        oh = jnp.where(jnp.logical_and(sel, iom == slot.astype(jnp.int32)),
                       1.0, 0.0)
        oh_k = oh * k
        oh_n = oh * nk

        coords = jnp.concatenate(
            [col(x1c, bj), col(y1c, bj), col(x2c, bj), col(y2c, bj)], axis=1)
        acc4 = acc4 + jnp.dot(oh, coords, preferred_element_type=f32,
                              precision=hi)
        acc1 = acc1 + (jnp.dot(oh_k, col(sc, bj), preferred_element_type=f32,
                               precision=hi)
                       - jnp.dot(oh_n, ones_col, preferred_element_type=f32,
                                 precision=hi))
        return (kept_before + jnp.sum(k), nk_before + jnp.sum(nk), acc4, acc1)

    init = (jnp.float32(0.0), jnp.float32(0.0),
            jnp.zeros((_MAX_DET, 4), f32), jnp.zeros((_MAX_DET, 1), f32))
    _, _, acc4, acc1 = jax.lax.fori_loop(0, nblk, sel_body, init)
    out_ref[:, :] = jnp.concatenate([acc4, acc1], axis=1)


def _pad_kernel(bref, sref, key_ref, x1_ref, y1_ref, x2_ref, y2_ref, s_ref):
    pad = _NPAD - _N
    s = sref[:, :]
    key_ref[pl.ds(0, _N), :] = -s
    key_ref[pl.ds(_N, pad), :] = jnp.full((pad, 1), 1.0, jnp.float32)
    s_ref[pl.ds(0, _N), :] = s
    s_ref[pl.ds(_N, pad), :] = jnp.full((pad, 1), -1.0, jnp.float32)
    zp = jnp.zeros((pad, 1), jnp.float32)
    for c, ref in enumerate((x1_ref, y1_ref, x2_ref, y2_ref)):
        ref[pl.ds(0, _N), :] = bref[:, c:c + 1]
        ref[pl.ds(_N, pad), :] = zp


def kernel(boxes, scores):
    cshape = jax.ShapeDtypeStruct((_NPAD, 1), jnp.float32)
    key, x1p, y1p, x2p, y2p, sp = pl.pallas_call(
        _pad_kernel,
        out_shape=[cshape] * 6,
    )(boxes, scores.reshape(_N, 1))

    _, x1, y1, x2, y2, s = jax.lax.sort(
        (key.reshape(_NPAD), x1p.reshape(_NPAD), y1p.reshape(_NPAD),
         x2p.reshape(_NPAD), y2p.reshape(_NPAD), sp.reshape(_NPAD)),
        dimension=0, num_keys=1, is_stable=True)

    return pl.pallas_call(
        _nms_kernel,
        out_shape=jax.ShapeDtypeStruct((_MAX_DET, 5), jnp.float32),
        scratch_shapes=[pltpu.VMEM((_NB, _B), jnp.float32)],
    )(x1.reshape(_NPAD, 1), y1.reshape(_NPAD, 1), x2.reshape(_NPAD, 1),
      y2.reshape(_NPAD, 1), s.reshape(_NPAD, 1))


# speedup vs baseline: 1.6832x; 1.6832x over previous
import jax
import jax.numpy as jnp
from jax.experimental import pallas as pl
from jax.experimental.pallas import tpu as pltpu

_N = 5000
_B = 256
_NB = (_N + _B - 1) // _B
_NPAD = _NB * _B
_SCORE_THRESH = 0.05
_NMS_THRESH = 0.5
_MAX_DET = 100


def _iou_mask(ax1, ay1, ax2, ay2, bx1, by1, bx2, by2):
    ix1 = jnp.maximum(ax1, bx1)
    iy1 = jnp.maximum(ay1, by1)
    ix2 = jnp.minimum(ax2, bx2)
    iy2 = jnp.minimum(ay2, by2)
    iw = jnp.maximum(ix2 - ix1, 0.0)
    ih = jnp.maximum(iy2 - iy1, 0.0)
    inter = iw * ih
    area_a = (ax2 - ax1) * (ay2 - ay1)
    area_b = (bx2 - bx1) * (by2 - by1)
    union = area_a + area_b - inter
    iou = inter / jnp.maximum(union, 1e-9)
    return iou > _NMS_THRESH


def _nms_kernel(x1c, y1c, x2c, y2c, sc,
                out_ref,
                keep_ref):
    f32 = jnp.float32
    hi = jax.lax.Precision.HIGHEST

    def row(ref, bi):
        return ref[pl.ds(bi, 1), :]

    def col(ref, bi):
        start = jnp.minimum(bi * _B, _N - _B)
        return ref[pl.ds(start, _B), :]

    ioj = jax.lax.broadcasted_iota(jnp.int32, (1, _B), 1)

    def owned_mask(bi):
        start = jnp.minimum(bi * _B, _N - _B)
        return (start + ioj) >= bi * _B

    ii = jax.lax.broadcasted_iota(jnp.int32, (_B, _B), 0)
    jj = jax.lax.broadcasted_iota(jnp.int32, (_B, _B), 1)
    upper = (ii < jj)
    upper_f = upper.astype(f32)
    eye = (ii == jj).astype(f32)

    def to_row(c):
        return jax.lax.dot_general(c, eye, (((0,), (0,)), ((), ())),
                                   preferred_element_type=f32, precision=hi)

    def matvec(k, m):
        return jnp.dot(k, m, preferred_element_type=f32)

    def main_cond(st):
        bi, cnt = st
        return jnp.logical_and(bi < _NB, cnt < jnp.float32(_MAX_DET))

    def main_body(st):
        bi, cnt = st
        bx1, by1 = to_row(col(x1c, bi)), to_row(col(y1c, bi))
        bx2, by2 = to_row(col(x2c, bi)), to_row(col(y2c, bi))
        sb = to_row(col(sc, bi))
        v = jnp.logical_and(sb > _SCORE_THRESH, owned_mask(bi)).astype(f32)

        def cross(bj, v):
            m = _iou_mask(col(x1c, bj), col(y1c, bj), col(x2c, bj), col(y2c, bj),
                          bx1, by1, bx2, by2)
            krow = row(keep_ref, bj)
            supp = matvec(krow, m.astype(f32))
            return jnp.where(supp > 0.0, 0.0, v)

        v = jax.lax.fori_loop(0, bi, cross, v)

        m = _iou_mask(col(x1c, bi), col(y1c, bi), col(x2c, bi), col(y2c, bi),
                      bx1, by1, bx2, by2)
        mf = jnp.where(upper, m.astype(f32), 0.0)
        vf = v

        def conv_cond(cs):
            _, changed = cs
            return changed

        def conv_body(cs):
            k, _ = cs
            supp = matvec(k, mf) > 0.0
            k_new = jnp.where(supp, 0.0, vf)
            return k_new, jnp.any(k_new != k)

        k, _ = jax.lax.while_loop(conv_cond, conv_body, (vf, jnp.bool_(True)))

        keep_ref[pl.ds(bi, 1), :] = k
        return bi + 1, cnt + jnp.sum(k)

    nblk, cnt = jax.lax.while_loop(main_cond, main_body,
                                   (jnp.int32(0), jnp.float32(0.0)))

    kcap = jnp.minimum(cnt, jnp.float32(_MAX_DET))
    iom = jax.lax.broadcasted_iota(jnp.int32, (_MAX_DET, _B), 0)
    ones_col = jnp.ones((_B, 1), f32)

    def sel_body(bj, carry):
        kept_before, nk_before, acc4, acc1 = carry
        k = row(keep_ref, bj)
        nk = (1.0 - k) * owned_mask(bj).astype(f32)

        pk = matvec(k, upper_f)
        pn = matvec(nk, upper_f)
        slot = jnp.where(k > 0.0, kept_before + pk, kcap + nk_before + pn)
        sel = jnp.logical_and(jnp.logical_or(k > 0.0, nk > 0.0),
                              slot < jnp.float32(_MAX_DET))
        oh = jnp.where(jnp.logical_and(sel, iom == slot.astype(jnp.int32)),
                       1.0, 0.0)
        oh_k = oh * k
        oh_n = oh * nk

        coords = jnp.concatenate(
            [col(x1c, bj), col(y1c, bj), col(x2c, bj), col(y2c, bj)], axis=1)
        acc4 = acc4 + jnp.dot(oh, coords, preferred_element_type=f32,
                              precision=hi)
        acc1 = acc1 + (jnp.dot(oh_k, col(sc, bj), preferred_element_type=f32,
                               precision=hi)
                       - jnp.dot(oh_n, ones_col, preferred_element_type=f32,
                                 precision=hi))
        return (kept_before + jnp.sum(k), nk_before + jnp.sum(nk), acc4, acc1)

    init = (jnp.float32(0.0), jnp.float32(0.0),
            jnp.zeros((_MAX_DET, 4), f32), jnp.zeros((_MAX_DET, 1), f32))
    _, _, acc4, acc1 = jax.lax.fori_loop(0, nblk, sel_body, init)
    out_ref[:, :] = jnp.concatenate([acc4, acc1], axis=1)


def kernel(boxes, scores):
    _, x1, y1, x2, y2, s = jax.lax.sort(
        (-scores, boxes[:, 0], boxes[:, 1], boxes[:, 2], boxes[:, 3],
         scores), dimension=0, num_keys=1, is_stable=True)

    return pl.pallas_call(
        _nms_kernel,
        out_shape=jax.ShapeDtypeStruct((_MAX_DET, 5), jnp.float32),
        scratch_shapes=[pltpu.VMEM((_NB, _B), jnp.float32)],
    )(x1.reshape(_N, 1), y1.reshape(_N, 1), x2.reshape(_N, 1),
      y2.reshape(_N, 1), s.reshape(_N, 1))


# speedup vs baseline: 1.9144x; 1.1373x over previous
import jax
import jax.numpy as jnp
from jax.experimental import pallas as pl
from jax.experimental.pallas import tpu as pltpu

_N = 5000
_B = 256
_NB = (_N + _B - 1) // _B
_SCORE_THRESH = 0.05
_NMS_THRESH = 0.5
_MAX_DET = 100


def _iou_mask(ax1, ay1, ax2, ay2, bx1, by1, bx2, by2):
    ix1 = jnp.maximum(ax1, bx1)
    iy1 = jnp.maximum(ay1, by1)
    ix2 = jnp.minimum(ax2, bx2)
    iy2 = jnp.minimum(ay2, by2)
    iw = jnp.maximum(ix2 - ix1, 0.0)
    ih = jnp.maximum(iy2 - iy1, 0.0)
    inter = iw * ih
    area_a = (ax2 - ax1) * (ay2 - ay1)
    area_b = (bx2 - bx1) * (by2 - by1)
    union = area_a + area_b - inter
    iou = inter / jnp.maximum(union, 1e-9)
    return iou > _NMS_THRESH


def _nms_kernel(bref,
                keyc,
                idxc,
                out_ref,
                keep_ref,
                sbox_ref):
    f32 = jnp.float32
    hi = jax.lax.Precision.HIGHEST

    def start_of(bi):
        return jnp.minimum(bi * _B, _N - _B)

    def row(ref, bi):
        return ref[pl.ds(bi, 1), :]

    def col(ref, bi):
        return ref[pl.ds(start_of(bi), _B), :]

    ioj = jax.lax.broadcasted_iota(jnp.int32, (1, _B), 1)

    def owned_mask(bi):
        return (start_of(bi) + ioj) >= bi * _B

    ii = jax.lax.broadcasted_iota(jnp.int32, (_B, _B), 0)
    jj = jax.lax.broadcasted_iota(jnp.int32, (_B, _B), 1)
    upper = (ii < jj)
    upper_f = upper.astype(f32)
    eye = (ii == jj).astype(f32)
    ion = jax.lax.broadcasted_iota(jnp.int32, (_B, _N), 1)

    def to_row(c):
        return jax.lax.dot_general(c, eye, (((0,), (0,)), ((), ())),
                                   preferred_element_type=f32, precision=hi)

    def matvec(k, m):
        return jnp.dot(k, m, preferred_element_type=f32)

    def main_cond(st):
        bi, cnt = st
        return jnp.logical_and(bi < _NB, cnt < jnp.float32(_MAX_DET))

    def main_body(st):
        bi, cnt = st
        gidx = col(idxc, bi)
        g = (ion == gidx).astype(f32)
        bcols = jnp.dot(g, bref[:, :], preferred_element_type=f32,
                        precision=hi)
        sbox_ref[pl.ds(start_of(bi), _B), :] = bcols

        ax1, ay1 = bcols[:, 0:1], bcols[:, 1:2]
        ax2, ay2 = bcols[:, 2:3], bcols[:, 3:4]
        bx1, by1, bx2, by2 = to_row(ax1), to_row(ay1), to_row(ax2), to_row(ay2)
        sb = -to_row(col(keyc, bi))
        v = jnp.logical_and(sb > _SCORE_THRESH, owned_mask(bi)).astype(f32)

        def cross(bj, v):
            sbb = sbox_ref[pl.ds(bj * _B, _B), :]
            m = _iou_mask(sbb[:, 0:1], sbb[:, 1:2], sbb[:, 2:3], sbb[:, 3:4],
                          bx1, by1, bx2, by2)
            krow = row(keep_ref, bj)
            supp = matvec(krow, m.astype(f32))
            return jnp.where(supp > 0.0, 0.0, v)

        v = jax.lax.fori_loop(0, bi, cross, v)

        m = _iou_mask(ax1, ay1, ax2, ay2, bx1, by1, bx2, by2)
        mf = jnp.where(upper, m.astype(f32), 0.0)
        vf = v

        def conv_cond(cs):
            _, changed = cs
            return changed

        def conv_body(cs):
            k, _ = cs
            supp = matvec(k, mf) > 0.0
            k_new = jnp.where(supp, 0.0, vf)
            return k_new, jnp.any(k_new != k)

        k, _ = jax.lax.while_loop(conv_cond, conv_body, (vf, jnp.bool_(True)))

        keep_ref[pl.ds(bi, 1), :] = k
        return bi + 1, cnt + jnp.sum(k)

    nblk, cnt = jax.lax.while_loop(main_cond, main_body,
                                   (jnp.int32(0), jnp.float32(0.0)))

    kcap = jnp.minimum(cnt, jnp.float32(_MAX_DET))
    iom = jax.lax.broadcasted_iota(jnp.int32, (_MAX_DET, _B), 0)
    ones_col = jnp.ones((_B, 1), f32)

    def sel_body(bj, carry):
        kept_before, nk_before, acc4, acc1 = carry
        k = row(keep_ref, bj)
        nk = (1.0 - k) * owned_mask(bj).astype(f32)

        pk = matvec(k, upper_f)
        pn = matvec(nk, upper_f)
        slot = jnp.where(k > 0.0, kept_before + pk, kcap + nk_before + pn)
        sel = jnp.logical_and(jnp.logical_or(k > 0.0, nk > 0.0),
                              slot < jnp.float32(_MAX_DET))
        oh = jnp.where(jnp.logical_and(sel, iom == slot.astype(jnp.int32)),
                       1.0, 0.0)
        oh_k = oh * k
        oh_n = oh * nk

        coords = sbox_ref[pl.ds(start_of(bj), _B), :]
        scol = 0.0 - col(keyc, bj)
        acc4 = acc4 + jnp.dot(oh, coords, preferred_element_type=f32,
                              precision=hi)
        acc1 = acc1 + (jnp.dot(oh_k, scol, preferred_element_type=f32,
                               precision=hi)
                       - jnp.dot(oh_n, ones_col, preferred_element_type=f32,
                                 precision=hi))
        return (kept_before + jnp.sum(k), nk_before + jnp.sum(nk), acc4, acc1)

    init = (jnp.float32(0.0), jnp.float32(0.0),
            jnp.zeros((_MAX_DET, 4), f32), jnp.zeros((_MAX_DET, 1), f32))
    _, _, acc4, acc1 = jax.lax.fori_loop(0, nblk, sel_body, init)
    out_ref[:, :] = jnp.concatenate([acc4, acc1], axis=1)


def kernel(boxes, scores):
    key, idx = jax.lax.sort(
        (-scores, jax.lax.iota(jnp.int32, _N)),
        dimension=0, num_keys=1, is_stable=True)

    return pl.pallas_call(
        _nms_kernel,
        out_shape=jax.ShapeDtypeStruct((_MAX_DET, 5), jnp.float32),
        scratch_shapes=[pltpu.VMEM((_NB, _B), jnp.float32),
                        pltpu.VMEM((_N, 4), jnp.float32)],
    )(boxes, key.reshape(_N, 1), idx.reshape(_N, 1))


# speedup vs baseline: 2.1797x; 1.1386x over previous
import jax
import jax.numpy as jnp
from jax.experimental import pallas as pl
from jax.experimental.pallas import tpu as pltpu

_N = 5000
_B = 128
_NB = (_N + _B - 1) // _B
_SCORE_THRESH = 0.05
_NMS_THRESH = 0.5
_MAX_DET = 100


def _iou_mask(ax1, ay1, ax2, ay2, bx1, by1, bx2, by2):
    ix1 = jnp.maximum(ax1, bx1)
    iy1 = jnp.maximum(ay1, by1)
    ix2 = jnp.minimum(ax2, bx2)
    iy2 = jnp.minimum(ay2, by2)
    iw = jnp.maximum(ix2 - ix1, 0.0)
    ih = jnp.maximum(iy2 - iy1, 0.0)
    inter = iw * ih
    area_a = (ax2 - ax1) * (ay2 - ay1)
    area_b = (bx2 - bx1) * (by2 - by1)
    union = area_a + area_b - inter
    iou = inter / jnp.maximum(union, 1e-9)
    return iou > _NMS_THRESH


def _nms_kernel(bref,
                keyc,
                idxc,
                out_ref,
                keep_ref,
                sbox_ref):
    f32 = jnp.float32
    hi = jax.lax.Precision.HIGHEST

    def start_of(bi):
        return jnp.minimum(bi * _B, _N - _B)

    def row(ref, bi):
        return ref[pl.ds(bi, 1), :]

    def col(ref, bi):
        return ref[pl.ds(start_of(bi), _B), :]

    ioj = jax.lax.broadcasted_iota(jnp.int32, (1, _B), 1)

    def owned_mask(bi):
        return (start_of(bi) + ioj) >= bi * _B

    ii = jax.lax.broadcasted_iota(jnp.int32, (_B, _B), 0)
    jj = jax.lax.broadcasted_iota(jnp.int32, (_B, _B), 1)
    upper = (ii < jj)
    upper_f = upper.astype(f32)
    eye = (ii == jj).astype(f32)
    ion = jax.lax.broadcasted_iota(jnp.int32, (_B, _N), 1)

    def to_row(c):
        return jax.lax.dot_general(c, eye, (((0,), (0,)), ((), ())),
                                   preferred_element_type=f32, precision=hi)

    def matvec(k, m):
        return jnp.dot(k, m, preferred_element_type=f32)

    def main_cond(st):
        bi, cnt = st
        return jnp.logical_and(bi < _NB, cnt < jnp.float32(_MAX_DET))

    def main_body(st):
        bi, cnt = st
        gidx = col(idxc, bi)
        g = (ion == gidx).astype(f32)
        bcols = jnp.dot(g, bref[:, :], preferred_element_type=f32,
                        precision=hi)
        sbox_ref[pl.ds(start_of(bi), _B), :] = bcols

        ax1, ay1 = bcols[:, 0:1], bcols[:, 1:2]
        ax2, ay2 = bcols[:, 2:3], bcols[:, 3:4]
        bx1, by1, bx2, by2 = to_row(ax1), to_row(ay1), to_row(ax2), to_row(ay2)
        sb = -to_row(col(keyc, bi))
        v = jnp.logical_and(sb > _SCORE_THRESH, owned_mask(bi)).astype(f32)

        def cross(bj, v):
            sbb = sbox_ref[pl.ds(bj * _B, _B), :]
            m = _iou_mask(sbb[:, 0:1], sbb[:, 1:2], sbb[:, 2:3], sbb[:, 3:4],
                          bx1, by1, bx2, by2)
            krow = row(keep_ref, bj)
            supp = matvec(krow, m.astype(f32))
            return jnp.where(supp > 0.0, 0.0, v)

        v = jax.lax.fori_loop(0, bi, cross, v)

        m = _iou_mask(ax1, ay1, ax2, ay2, bx1, by1, bx2, by2)
        mf = jnp.where(upper, m.astype(f32), 0.0)
        vf = v

        def conv_cond(cs):
            _, changed = cs
            return changed

        def conv_body(cs):
            k, _ = cs
            supp = matvec(k, mf) > 0.0
            k_new = jnp.where(supp, 0.0, vf)
            return k_new, jnp.any(k_new != k)

        k, _ = jax.lax.while_loop(conv_cond, conv_body, (vf, jnp.bool_(True)))

        keep_ref[pl.ds(bi, 1), :] = k
        return bi + 1, cnt + jnp.sum(k)

    nblk, cnt = jax.lax.while_loop(main_cond, main_body,
                                   (jnp.int32(0), jnp.float32(0.0)))

    kcap = jnp.minimum(cnt, jnp.float32(_MAX_DET))
    iom = jax.lax.broadcasted_iota(jnp.int32, (_MAX_DET, _B), 0)
    ones_col = jnp.ones((_B, 1), f32)

    def sel_body(bj, carry):
        kept_before, nk_before, acc4, acc1 = carry
        k = row(keep_ref, bj)
        nk = (1.0 - k) * owned_mask(bj).astype(f32)

        pk = matvec(k, upper_f)
        pn = matvec(nk, upper_f)
        slot = jnp.where(k > 0.0, kept_before + pk, kcap + nk_before + pn)
        sel = jnp.logical_and(jnp.logical_or(k > 0.0, nk > 0.0),
                              slot < jnp.float32(_MAX_DET))
        oh = jnp.where(jnp.logical_and(sel, iom == slot.astype(jnp.int32)),
                       1.0, 0.0)
        oh_k = oh * k
        oh_n = oh * nk

        coords = sbox_ref[pl.ds(start_of(bj), _B), :]
        scol = 0.0 - col(keyc, bj)
        acc4 = acc4 + jnp.dot(oh, coords, preferred_element_type=f32,
                              precision=hi)
        acc1 = acc1 + (jnp.dot(oh_k, scol, preferred_element_type=f32,
                               precision=hi)
                       - jnp.dot(oh_n, ones_col, preferred_element_type=f32,
                                 precision=hi))
        return (kept_before + jnp.sum(k), nk_before + jnp.sum(nk), acc4, acc1)

    init = (jnp.float32(0.0), jnp.float32(0.0),
            jnp.zeros((_MAX_DET, 4), f32), jnp.zeros((_MAX_DET, 1), f32))
    _, _, acc4, acc1 = jax.lax.fori_loop(0, nblk, sel_body, init)
    out_ref[:, :] = jnp.concatenate([acc4, acc1], axis=1)


def kernel(boxes, scores):
    key, idx = jax.lax.sort(
        (-scores, jax.lax.iota(jnp.int32, _N)),
        dimension=0, num_keys=1, is_stable=True)

    return pl.pallas_call(
        _nms_kernel,
        out_shape=jax.ShapeDtypeStruct((_MAX_DET, 5), jnp.float32),
        scratch_shapes=[pltpu.VMEM((_NB, _B), jnp.float32),
                        pltpu.VMEM((_N, 4), jnp.float32)],
    )(boxes, key.reshape(_N, 1), idx.reshape(_N, 1))


# speedup vs baseline: 2.2036x; 1.0110x over previous
import jax
import jax.numpy as jnp
from jax.experimental import pallas as pl
from jax.experimental.pallas import tpu as pltpu

_N = 5000
_B = 128
_NB = (_N + _B - 1) // _B
_SCORE_THRESH = 0.05
_NMS_THRESH = 0.5
_MAX_DET = 100


def _iou_mask(ax1, ay1, ax2, ay2, bx1, by1, bx2, by2):
    ix1 = jnp.maximum(ax1, bx1)
    iy1 = jnp.maximum(ay1, by1)
    ix2 = jnp.minimum(ax2, bx2)
    iy2 = jnp.minimum(ay2, by2)
    iw = jnp.maximum(ix2 - ix1, 0.0)
    ih = jnp.maximum(iy2 - iy1, 0.0)
    inter = iw * ih
    area_a = (ax2 - ax1) * (ay2 - ay1)
    area_b = (bx2 - bx1) * (by2 - by1)
    union = area_a + area_b - inter
    iou = inter / jnp.maximum(union, 1e-9)
    return iou > _NMS_THRESH


def _nms_kernel(bref,
                ssc,
                idxc,
                out_ref,
                keep_ref,
                sbox_ref):
    f32 = jnp.float32
    hi = jax.lax.Precision.HIGHEST

    def start_of(bi):
        return jnp.minimum(bi * _B, _N - _B)

    def row(ref, bi):
        return ref[pl.ds(bi, 1), :]

    def col(ref, bi):
        return ref[pl.ds(start_of(bi), _B), :]

    ioj = jax.lax.broadcasted_iota(jnp.int32, (1, _B), 1)

    def owned_mask(bi):
        return (start_of(bi) + ioj) >= bi * _B

    ii = jax.lax.broadcasted_iota(jnp.int32, (_B, _B), 0)
    jj = jax.lax.broadcasted_iota(jnp.int32, (_B, _B), 1)
    upper = (ii < jj)
    upper_f = upper.astype(f32)
    eye = (ii == jj).astype(f32)
    ion = jax.lax.broadcasted_iota(jnp.int32, (_B, _N), 1)

    def to_row(c):
        return jax.lax.dot_general(c, eye, (((0,), (0,)), ((), ())),
                                   preferred_element_type=f32, precision=hi)

    def matvec(k, m):
        return jnp.dot(k, m, preferred_element_type=f32)

    def main_cond(st):
        bi, cnt = st
        return jnp.logical_and(bi < _NB, cnt < jnp.float32(_MAX_DET))

    def main_body(st):
        bi, cnt = st
        gidx = col(idxc, bi)
        g = (ion == gidx).astype(f32)
        bcols = jnp.dot(g, bref[:, :], preferred_element_type=f32,
                        precision=hi)
        sbox_ref[pl.ds(start_of(bi), _B), :] = bcols

        ax1, ay1 = bcols[:, 0:1], bcols[:, 1:2]
        ax2, ay2 = bcols[:, 2:3], bcols[:, 3:4]
        bx1, by1, bx2, by2 = to_row(ax1), to_row(ay1), to_row(ax2), to_row(ay2)
        sb = to_row(col(ssc, bi))
        v = jnp.logical_and(sb > _SCORE_THRESH, owned_mask(bi)).astype(f32)

        def cross(bj, v):
            sbb = sbox_ref[pl.ds(bj * _B, _B), :]
            m = _iou_mask(sbb[:, 0:1], sbb[:, 1:2], sbb[:, 2:3], sbb[:, 3:4],
                          bx1, by1, bx2, by2)
            krow = row(keep_ref, bj)
            supp = matvec(krow, m.astype(f32))
            return jnp.where(supp > 0.0, 0.0, v)

        v = jax.lax.fori_loop(0, bi, cross, v)

        m = _iou_mask(ax1, ay1, ax2, ay2, bx1, by1, bx2, by2)
        mf = jnp.where(upper, m.astype(f32), 0.0)
        vf = v

        def conv_cond(cs):
            _, changed = cs
            return changed

        def conv_body(cs):
            k, _ = cs
            supp = matvec(k, mf) > 0.0
            k_new = jnp.where(supp, 0.0, vf)
            return k_new, jnp.any(k_new != k)

        k, _ = jax.lax.while_loop(conv_cond, conv_body, (vf, jnp.bool_(True)))

        keep_ref[pl.ds(bi, 1), :] = k
        return bi + 1, cnt + jnp.sum(k)

    nblk, cnt = jax.lax.while_loop(main_cond, main_body,
                                   (jnp.int32(0), jnp.float32(0.0)))

    kcap = jnp.minimum(cnt, jnp.float32(_MAX_DET))
    iom = jax.lax.broadcasted_iota(jnp.int32, (_MAX_DET, _B), 0)
    ones_col = jnp.ones((_B, 1), f32)

    def sel_body(bj, carry):
        kept_before, nk_before, acc4, acc1 = carry
        k = row(keep_ref, bj)
        nk = (1.0 - k) * owned_mask(bj).astype(f32)

        pk = matvec(k, upper_f)
        pn = matvec(nk, upper_f)
        slot = jnp.where(k > 0.0, kept_before + pk, kcap + nk_before + pn)
        sel = jnp.logical_and(jnp.logical_or(k > 0.0, nk > 0.0),
                              slot < jnp.float32(_MAX_DET))
        oh = jnp.where(jnp.logical_and(sel, iom == slot.astype(jnp.int32)),
                       1.0, 0.0)
        oh_k = oh * k
        oh_n = oh * nk

        coords = sbox_ref[pl.ds(start_of(bj), _B), :]
        scol = col(ssc, bj)
        acc4 = acc4 + jnp.dot(oh, coords, preferred_element_type=f32,
                              precision=hi)
        acc1 = acc1 + (jnp.dot(oh_k, scol, preferred_element_type=f32,
                               precision=hi)
                       - jnp.dot(oh_n, ones_col, preferred_element_type=f32,
                                 precision=hi))
        return (kept_before + jnp.sum(k), nk_before + jnp.sum(nk), acc4, acc1)

    init = (jnp.float32(0.0), jnp.float32(0.0),
            jnp.zeros((_MAX_DET, 4), f32), jnp.zeros((_MAX_DET, 1), f32))
    _, _, acc4, acc1 = jax.lax.fori_loop(0, nblk, sel_body, init)
    out_ref[:, :] = jnp.concatenate([acc4, acc1], axis=1)


def kernel(boxes, scores):
    svals, idx = jax.lax.top_k(scores, _N)

    return pl.pallas_call(
        _nms_kernel,
        out_shape=jax.ShapeDtypeStruct((_MAX_DET, 5), jnp.float32),
        scratch_shapes=[pltpu.VMEM((_NB, _B), jnp.float32),
                        pltpu.VMEM((_N, 4), jnp.float32)],
    )(boxes, svals.reshape(_N, 1), idx.reshape(_N, 1))


# speedup vs baseline: 2.6897x; 1.2206x over previous
import jax
import jax.numpy as jnp
from jax.experimental import pallas as pl
from jax.experimental.pallas import tpu as pltpu

_N = 5000
_B = 128
_NB = (_N + _B - 1) // _B
_NPAD = _NB * _B
_SCORE_THRESH = 0.05
_NMS_THRESH = 0.5
_MAX_DET = 100


def _iou_mask(ax1, ay1, ax2, ay2, bx1, by1, bx2, by2):
    ix1 = jnp.maximum(ax1, bx1)
    iy1 = jnp.maximum(ay1, by1)
    ix2 = jnp.minimum(ax2, bx2)
    iy2 = jnp.minimum(ay2, by2)
    iw = jnp.maximum(ix2 - ix1, 0.0)
    ih = jnp.maximum(iy2 - iy1, 0.0)
    inter = iw * ih
    area_a = (ax2 - ax1) * (ay2 - ay1)
    area_b = (bx2 - bx1) * (by2 - by1)
    union = area_a + area_b - inter
    iou = inter / jnp.maximum(union, 1e-9)
    return iou > _NMS_THRESH


def _nms_kernel(bref,
                ssr,
                idxr,
                out_ref,
                keep_ref,
                sbox_ref):
    f32 = jnp.float32
    hi = jax.lax.Precision.HIGHEST

    def row(ref, bi):
        return ref[pl.ds(bi, 1), :]

    rowblk = row

    ioj = jax.lax.broadcasted_iota(jnp.int32, (1, _B), 1)

    def owned_mask(bi):
        return (bi * _B + ioj) < _N

    ii = jax.lax.broadcasted_iota(jnp.int32, (_B, _B), 0)
    jj = jax.lax.broadcasted_iota(jnp.int32, (_B, _B), 1)
    upper = (ii < jj)
    upper_f = upper.astype(f32)
    eye = (ii == jj).astype(f32)
    ion = jax.lax.broadcasted_iota(jnp.int32, (_N, _B), 0)

    def to_row(c):
        return jax.lax.dot_general(c, eye, (((0,), (0,)), ((), ())),
                                   preferred_element_type=f32, precision=hi)

    def to_col(r):
        return jax.lax.dot_general(eye, r, (((1,), (1,)), ((), ())),
                                   preferred_element_type=f32, precision=hi)

    def matvec(k, m):
        return jnp.dot(k, m, preferred_element_type=f32)

    def main_cond(st):
        bi, cnt = st
        return jnp.logical_and(bi < _NB, cnt < jnp.float32(_MAX_DET))

    def main_body(st):
        bi, cnt = st
        gidx = rowblk(idxr, bi)
        g = (ion == gidx).astype(f32)
        bcols = jax.lax.dot_general(g, bref[:, :], (((0,), (0,)), ((), ())),
                                    preferred_element_type=f32,
                                    precision=hi)
        sbox_ref[pl.ds(bi * _B, _B), :] = bcols

        ax1, ay1 = bcols[:, 0:1], bcols[:, 1:2]
        ax2, ay2 = bcols[:, 2:3], bcols[:, 3:4]
        bx1, by1, bx2, by2 = to_row(ax1), to_row(ay1), to_row(ax2), to_row(ay2)
        sb = rowblk(ssr, bi)
        v = jnp.logical_and(sb > _SCORE_THRESH, owned_mask(bi)).astype(f32)

        def cross(bj, v):
            sbb = sbox_ref[pl.ds(bj * _B, _B), :]
            m = _iou_mask(sbb[:, 0:1], sbb[:, 1:2], sbb[:, 2:3], sbb[:, 3:4],
                          bx1, by1, bx2, by2)
            krow = row(keep_ref, bj)
            supp = matvec(krow, m.astype(f32))
            return jnp.where(supp > 0.0, 0.0, v)

        v = jax.lax.fori_loop(0, bi, cross, v)

        m = _iou_mask(ax1, ay1, ax2, ay2, bx1, by1, bx2, by2)
        mf = jnp.where(upper, m.astype(f32), 0.0)
        vf = v

        def conv_cond(cs):
            _, changed = cs
            return changed

        def conv_body(cs):
            k, _ = cs
            supp = matvec(k, mf) > 0.0
            k_new = jnp.where(supp, 0.0, vf)
            return k_new, jnp.any(k_new != k)

        k, _ = jax.lax.while_loop(conv_cond, conv_body, (vf, jnp.bool_(True)))

        keep_ref[pl.ds(bi, 1), :] = k
        return bi + 1, cnt + jnp.sum(k)

    nblk, cnt = jax.lax.while_loop(main_cond, main_body,
                                   (jnp.int32(0), jnp.float32(0.0)))

    kcap = jnp.minimum(cnt, jnp.float32(_MAX_DET))
    iom = jax.lax.broadcasted_iota(jnp.int32, (_MAX_DET, _B), 0)
    ones_col = jnp.ones((_B, 1), f32)

    def sel_body(bj, carry):
        kept_before, nk_before, acc4, acc1 = carry
        k = row(keep_ref, bj)
        nk = (1.0 - k) * owned_mask(bj).astype(f32)

        pk = matvec(k, upper_f)
        pn = matvec(nk, upper_f)
        slot = jnp.where(k > 0.0, kept_before + pk, kcap + nk_before + pn)
        sel = jnp.logical_and(jnp.logical_or(k > 0.0, nk > 0.0),
                              slot < jnp.float32(_MAX_DET))
        oh = jnp.where(jnp.logical_and(sel, iom == slot.astype(jnp.int32)),
                       1.0, 0.0)
        oh_k = oh * k
        oh_n = oh * nk

        coords = sbox_ref[pl.ds(bj * _B, _B), :]
        scol = to_col(rowblk(ssr, bj))
        acc4 = acc4 + jnp.dot(oh, coords, preferred_element_type=f32,
                              precision=hi)
        acc1 = acc1 + (jnp.dot(oh_k, scol, preferred_element_type=f32,
                               precision=hi)
                       - jnp.dot(oh_n, ones_col, preferred_element_type=f32,
                                 precision=hi))
        return (kept_before + jnp.sum(k), nk_before + jnp.sum(nk), acc4, acc1)

    init = (jnp.float32(0.0), jnp.float32(0.0),
            jnp.zeros((_MAX_DET, 4), f32), jnp.zeros((_MAX_DET, 1), f32))
    _, _, acc4, acc1 = jax.lax.fori_loop(0, nblk, sel_body, init)
    out_ref[:, :] = jnp.concatenate([acc4, acc1], axis=1)


def kernel(boxes, scores):
    scores_p = jnp.concatenate(
        [scores, jnp.full((_NPAD - _N,), -1.0, jnp.float32)])
    svals, idx = jax.lax.top_k(scores_p, _NPAD)

    return pl.pallas_call(
        _nms_kernel,
        out_shape=jax.ShapeDtypeStruct((_MAX_DET, 5), jnp.float32),
        scratch_shapes=[pltpu.VMEM((_NB, _B), jnp.float32),
                        pltpu.VMEM((_NPAD, 4), jnp.float32)],
    )(boxes, svals.reshape(_NB, _B), idx.reshape(_NB, _B))
